# baseline (speedup 1.0000x reference)
import functools

import jax
import jax.numpy as jnp
from jax import lax
from jax.experimental import pallas as pl
from jax.experimental.pallas import tpu as pltpu
from jax.experimental.pallas import tpu_sc as plsc

NC = 2
NS = 16
NW = NC * NS
LANES = 128
CK = 8


def _mesh():
    return plsc.VectorSubcoreMesh(core_axis_name="c", subcore_axis_name="s")


def _sc_degree(idx2d, n_pad):
    rows = idx2d.shape[0]
    per_tile = n_pad // NS
    chunks_per_w = rows // CK // NW
    n_iter = chunks_per_w // 2

    @functools.partial(
        pl.kernel,
        out_type=jax.ShapeDtypeStruct((NC, n_pad), jnp.float32),
        mesh=_mesh(),
        scratch_types=[
            pltpu.VMEM((per_tile,), jnp.float32),
            pltpu.VMEM((2, CK, LANES), jnp.int32),
            pltpu.VMEM((LANES,), jnp.float32),
            pltpu.VMEM_SHARED((n_pad,), jnp.float32),
        ] + [pltpu.SemaphoreType.DMA] * 4,
    )
    def k(idx_hbm, out_hbm, buf_v, idx_v, ones_v, deg_sh, si0, si1, ss0, ss1):
        sem_i = (si0, si1)
        sem_s = (ss0, ss1)
        cid = lax.axis_index("c")
        sid = lax.axis_index("s")
        wid = sid * NC + cid

        @pl.loop(0, per_tile, step=16)
        def _(i):
            buf_v[pl.ds(i, 16)] = jnp.zeros((16,), jnp.float32)

        @pl.loop(0, LANES, step=16)
        def _(i):
            ones_v[pl.ds(i, 16)] = jnp.ones((16,), jnp.float32)

        pltpu.sync_copy(buf_v, deg_sh.at[pl.ds(sid * per_tile, per_tile)])
        plsc.subcore_barrier()

        def fire_idx(t, slot):
            base = (wid + (2 * t + slot) * NW) * CK
            pltpu.async_copy(idx_hbm.at[pl.ds(base, CK)], idx_v.at[slot],
                             sem_i[slot])

        def wait_idx(slot):
            pltpu.make_async_copy(idx_hbm.at[pl.ds(0, CK)], idx_v.at[slot],
                                  sem_i[slot]).wait()

        def fire_scatters(slot):
            for j in range(CK):
                pltpu.async_copy(ones_v, deg_sh.at[idx_v.at[slot].at[j]],
                                 sem_s[slot], add=True)

        def drain_scatters(slot):
            for j in range(CK):
                pltpu.make_async_copy(ones_v, deg_sh.at[idx_v.at[slot].at[j]],
                                      sem_s[slot]).wait()

        fire_idx(0, 0)

        @pl.loop(0, n_iter)
        def _(t):
            fire_idx(t, 1)
            wait_idx(0)
            fire_scatters(0)
            wait_idx(1)
            fire_scatters(1)
            drain_scatters(0)

            @pl.when(t < n_iter - 1)
            def _():
                fire_idx(t + 1, 0)

            drain_scatters(1)

        plsc.subcore_barrier()
        pltpu.sync_copy(deg_sh.at[pl.ds(sid * per_tile, per_tile)], buf_v)
        pltpu.sync_copy(buf_v, out_hbm.at[cid].at[pl.ds(sid * per_tile, per_tile)])

    return k(idx2d)


def _sc_scatter(z_pad, src2d, dst2d, n_pad):
    rows, EL = src2d.shape
    d = z_pad.shape[1]
    per_tile = n_pad // NS
    chunks_per_w = rows // CK // NW
    n_iter = chunks_per_w // 2

    NBUF = 8
    DEPTH = 7
    NU = 4 * CK

    @functools.partial(
        pl.kernel,
        out_type=jax.ShapeDtypeStruct((NC, n_pad, d), jnp.float32),
        mesh=_mesh(),
        scratch_types=[
            pltpu.VMEM((2, CK, EL), jnp.int32),
            pltpu.VMEM((2, CK, EL), jnp.int32),
            pltpu.VMEM((NBUF, EL, d), jnp.float32),
            pltpu.VMEM_SHARED((n_pad, d), jnp.float32),
        ] + [pltpu.SemaphoreType.DMA] * (2 * NBUF + 2),
    )
    def k(z_hbm, src_hbm, dst_hbm, out_hbm, si_v, di_v, rows_v,
          acc_sh, *sems):
        sem_g = sems[:NBUF]
        sem_s = sems[NBUF:2 * NBUF]
        sem_i = sems[2 * NBUF:]
        cid = lax.axis_index("c")
        sid = lax.axis_index("s")
        wid = sid * NC + cid

        @pl.loop(0, EL)
        def _(r):
            @pl.loop(0, d, step=16)
            def _(c):
                rows_v[0, r, pl.ds(c, 16)] = jnp.zeros((16,), jnp.float32)

        @pl.loop(0, per_tile // EL)
        def _(j):
            pltpu.sync_copy(rows_v.at[0],
                            acc_sh.at[pl.ds(sid * per_tile + j * EL, EL)])

        plsc.subcore_barrier()

        def fire_idx(t, slot):
            base = (wid + (2 * t + slot) * NW) * CK
            pltpu.async_copy(src_hbm.at[pl.ds(base, CK)], si_v.at[slot],
                             sem_i[slot])
            pltpu.async_copy(dst_hbm.at[pl.ds(base, CK)], di_v.at[slot],
                             sem_i[slot])

        def wait_idx(slot):
            pltpu.make_async_copy(src_hbm.at[pl.ds(0, CK)], si_v.at[slot],
                                  sem_i[slot]).wait()
            pltpu.make_async_copy(dst_hbm.at[pl.ds(0, CK)], di_v.at[slot],
                                  sem_i[slot]).wait()

        def g_idx(u):
            slot, r = divmod(u, 2 * CK)
            return (si_v if u % 2 == 0 else di_v).at[slot].at[r // 2]

        def s_idx(u):
            slot, r = divmod(u, 2 * CK)
            return (di_v if u % 2 == 0 else si_v).at[slot].at[r // 2]

        def start_g(u):
            pltpu.async_copy(z_hbm.at[g_idx(u)], rows_v.at[u % NBUF],
                             sem_g[u % NBUF])

        def wait_g(u):
            pltpu.make_async_copy(z_hbm.at[g_idx(u)], rows_v.at[u % NBUF],
                                  sem_g[u % NBUF]).wait()

        def start_s(u):
            pltpu.async_copy(rows_v.at[u % NBUF], acc_sh.at[s_idx(u)],
                             sem_s[u % NBUF], add=True)

        def wait_s(u):
            pltpu.make_async_copy(rows_v.at[u % NBUF], acc_sh.at[s_idx(u)],
                                  sem_s[u % NBUF]).wait()

        fire_idx(0, 0)

        @pl.loop(0, n_iter)
        def _(t):
            fire_idx(t, 1)
            wait_idx(0)
            for u in range(DEPTH):
                start_g(u)
            for u in range(NU):
                if u == 2 * CK - DEPTH:
                    wait_idx(1)
                wait_g(u)
                start_s(u)
                if u + DEPTH < NU:
                    if u >= 1:
                        wait_s(u - 1)
                    start_g(u + DEPTH)
                if u == 2 * CK + 1:
                    @pl.when(t < n_iter - 1)
                    def _():
                        fire_idx(t + 1, 0)
            for u in range(NU - DEPTH - 1, NU):
                wait_s(u)

        plsc.subcore_barrier()

        @pl.loop(0, per_tile // EL)
        def _(j):
            base = sid * per_tile + j * EL
            pltpu.sync_copy(acc_sh.at[pl.ds(base, EL)], rows_v.at[0])
            pltpu.sync_copy(rows_v.at[0], out_hbm.at[cid].at[pl.ds(base, EL)])

    return k(z_pad, src2d, dst2d)


def _tc_matmul(x, w):
    n, _ = x.shape
    dout = w.shape[0]

    def body(x_ref, w_ref, y_ref):
        y_ref[...] = lax.dot_general(x_ref[...], w_ref[...],
                                     (((1,), (1,)), ((), ())),
                                     preferred_element_type=jnp.float32)

    return pl.pallas_call(
        body,
        out_shape=jax.ShapeDtypeStruct((n, dout), jnp.float32),
    )(x, w)


def _tc_scale(y, deg_parts):
    n, dout = y.shape
    n_pad = deg_parts.shape[1]

    def body(y_ref, dp_ref, z_ref, dis_ref):
        deg = dp_ref[0] + dp_ref[1] + 1.0
        dis = lax.rsqrt(deg)
        disn = dis[:n]
        z_ref[pl.ds(0, n)] = y_ref[...] * disn
        z_ref[pl.ds(n, n_pad - n)] = jnp.zeros((n_pad - n, dout), jnp.float32)
        dis_ref[...] = disn

    return pl.pallas_call(
        body,
        out_shape=(jax.ShapeDtypeStruct((n_pad, dout), jnp.float32),
                   jax.ShapeDtypeStruct((n, 1), jnp.float32)),
    )(y, deg_parts.reshape(NC, n_pad, 1))


def _tc_finish(s_parts, z_pad, dis, bias):
    n = dis.shape[0]
    d = z_pad.shape[1]

    def body(sp_ref, z_ref, dis_ref, b_ref, o_ref):
        s = sp_ref[0][:n] + sp_ref[1][:n] + z_ref[:n]
        o_ref[...] = jnp.maximum(s * dis_ref[...] + b_ref[...], 0.0)

    return pl.pallas_call(
        body,
        out_shape=jax.ShapeDtypeStruct((n, d), jnp.float32),
    )(s_parts, z_pad, dis, bias)


def kernel(x, edge_index, num_nodes, W, bias):
    n, _ = x.shape
    e = edge_index.shape[1]
    per_tile = -(-n // NS)
    per_tile = -(-per_tile // LANES) * LANES
    n_pad = per_tile * NS

    src = edge_index[0].astype(jnp.int32)
    dst = edge_index[1].astype(jnp.int32)

    def dead(k):
        return n + (jnp.arange(k, dtype=jnp.int32) % (n_pad - n))

    unit = LANES * NW * CK * 2
    hist_rows = (-(-2 * e // unit) * unit) // LANES
    hist_pad = hist_rows * LANES - 2 * e
    idx_all = jnp.concatenate([src, dst, dead(hist_pad)]).reshape(hist_rows, LANES)

    EL = 32
    eunit = EL * NW * CK * 2
    e_rows = (-(-e // eunit) * eunit) // EL
    e_pad = e_rows * EL - e
    src2d = jnp.concatenate([src, dead(e_pad)]).reshape(e_rows, EL)
    dst2d = jnp.concatenate([dst, dead(e_pad)]).reshape(e_rows, EL)

    y = _tc_matmul(x, W)
    deg_parts = _sc_degree(idx_all, n_pad)
    z_pad, dis = _tc_scale(y, deg_parts)
    s_parts = _sc_scatter(z_pad, src2d, dst2d, n_pad)
    return _tc_finish(s_parts, z_pad, dis, bias)


# speedup vs baseline: 1.0600x; 1.0600x over previous
import functools

import jax
import jax.numpy as jnp
from jax import lax
from jax.experimental import pallas as pl
from jax.experimental.pallas import tpu as pltpu
from jax.experimental.pallas import tpu_sc as plsc

NC = 2
NS = 16
NW = NC * NS
LANES = 128
CK = 16


def _mesh():
    return plsc.VectorSubcoreMesh(core_axis_name="c", subcore_axis_name="s")


def _sc_degree(idx2d, n_pad):
    rows = idx2d.shape[0]
    per_tile = n_pad // NS
    chunks_per_w = rows // CK // NW
    n_iter = chunks_per_w // 2

    @functools.partial(
        pl.kernel,
        out_type=jax.ShapeDtypeStruct((NC, n_pad), jnp.float32),
        mesh=_mesh(),
        scratch_types=[
            pltpu.VMEM((per_tile,), jnp.float32),
            pltpu.VMEM((2, CK, LANES), jnp.int32),
            pltpu.VMEM((LANES,), jnp.float32),
            pltpu.VMEM_SHARED((n_pad,), jnp.float32),
        ] + [pltpu.SemaphoreType.DMA] * 4,
    )
    def k(idx_hbm, out_hbm, buf_v, idx_v, ones_v, deg_sh, si0, si1, ss0, ss1):
        sem_i = (si0, si1)
        sem_s = (ss0, ss1)
        cid = lax.axis_index("c")
        sid = lax.axis_index("s")
        wid = sid * NC + cid

        @pl.loop(0, per_tile, step=16)
        def _(i):
            buf_v[pl.ds(i, 16)] = jnp.zeros((16,), jnp.float32)

        @pl.loop(0, LANES, step=16)
        def _(i):
            ones_v[pl.ds(i, 16)] = jnp.ones((16,), jnp.float32)

        pltpu.sync_copy(buf_v, deg_sh.at[pl.ds(sid * per_tile, per_tile)])
        plsc.subcore_barrier()

        def fire_idx(t, slot):
            base = (wid + (2 * t + slot) * NW) * CK
            pltpu.async_copy(idx_hbm.at[pl.ds(base, CK)], idx_v.at[slot],
                             sem_i[slot])

        def wait_idx(slot):
            pltpu.make_async_copy(idx_hbm.at[pl.ds(0, CK)], idx_v.at[slot],
                                  sem_i[slot]).wait()

        def fire_scatters(slot):
            for j in range(CK):
                pltpu.async_copy(ones_v, deg_sh.at[idx_v.at[slot].at[j]],
                                 sem_s[slot], add=True)

        def drain_scatters(slot):
            for j in range(CK):
                pltpu.make_async_copy(ones_v, deg_sh.at[idx_v.at[slot].at[j]],
                                      sem_s[slot]).wait()

        fire_idx(0, 0)

        @pl.loop(0, n_iter)
        def _(t):
            fire_idx(t, 1)
            wait_idx(0)
            fire_scatters(0)
            wait_idx(1)
            fire_scatters(1)
            drain_scatters(0)

            @pl.when(t < n_iter - 1)
            def _():
                fire_idx(t + 1, 0)

            drain_scatters(1)

        plsc.subcore_barrier()
        pltpu.sync_copy(deg_sh.at[pl.ds(sid * per_tile, per_tile)], buf_v)
        pltpu.sync_copy(buf_v, out_hbm.at[cid].at[pl.ds(sid * per_tile, per_tile)])

    return k(idx2d)


def _sc_scatter(z_pad, src2d, dst2d, n_pad):
    rows, EL = src2d.shape
    d = z_pad.shape[1]
    per_tile = n_pad // NS
    chunks_per_w = rows // CK // NW
    n_iter = chunks_per_w // 2

    NBUF = 5
    DEPTH = 4
    NU = 4 * CK

    @functools.partial(
        pl.kernel,
        out_type=jax.ShapeDtypeStruct((NC, n_pad, d), jnp.float32),
        mesh=_mesh(),
        scratch_types=[
            pltpu.VMEM((2, CK, EL), jnp.int32),
            pltpu.VMEM((2, CK, EL), jnp.int32),
            pltpu.VMEM((NBUF, EL, d), jnp.float32),
            pltpu.VMEM_SHARED((n_pad, d), jnp.float32),
        ] + [pltpu.SemaphoreType.DMA] * (2 * NBUF + 2),
    )
    def k(z_hbm, src_hbm, dst_hbm, out_hbm, si_v, di_v, rows_v,
          acc_sh, *sems):
        sem_g = sems[:NBUF]
        sem_s = sems[NBUF:2 * NBUF]
        sem_i = sems[2 * NBUF:]
        cid = lax.axis_index("c")
        sid = lax.axis_index("s")
        wid = sid * NC + cid

        @pl.loop(0, EL)
        def _(r):
            @pl.loop(0, d, step=16)
            def _(c):
                rows_v[0, r, pl.ds(c, 16)] = jnp.zeros((16,), jnp.float32)

        @pl.loop(0, per_tile // EL)
        def _(j):
            pltpu.sync_copy(rows_v.at[0],
                            acc_sh.at[pl.ds(sid * per_tile + j * EL, EL)])

        plsc.subcore_barrier()

        def fire_idx(t, slot):
            base = (wid + (2 * t + slot) * NW) * CK
            pltpu.async_copy(src_hbm.at[pl.ds(base, CK)], si_v.at[slot],
                             sem_i[slot])
            pltpu.async_copy(dst_hbm.at[pl.ds(base, CK)], di_v.at[slot],
                             sem_i[slot])

        def wait_idx(slot):
            pltpu.make_async_copy(src_hbm.at[pl.ds(0, CK)], si_v.at[slot],
                                  sem_i[slot]).wait()
            pltpu.make_async_copy(dst_hbm.at[pl.ds(0, CK)], di_v.at[slot],
                                  sem_i[slot]).wait()

        def g_idx(u):
            slot, r = divmod(u, 2 * CK)
            return (si_v if u % 2 == 0 else di_v).at[slot].at[r // 2]

        def s_idx(u):
            slot, r = divmod(u, 2 * CK)
            return (di_v if u % 2 == 0 else si_v).at[slot].at[r // 2]

        def start_g(u):
            pltpu.async_copy(z_hbm.at[g_idx(u)], rows_v.at[u % NBUF],
                             sem_g[u % NBUF])

        def wait_g(u):
            pltpu.make_async_copy(z_hbm.at[g_idx(u)], rows_v.at[u % NBUF],
                                  sem_g[u % NBUF]).wait()

        def start_s(u):
            pltpu.async_copy(rows_v.at[u % NBUF], acc_sh.at[s_idx(u)],
                             sem_s[u % NBUF], add=True)

        def wait_s(u):
            pltpu.make_async_copy(rows_v.at[u % NBUF], acc_sh.at[s_idx(u)],
                                  sem_s[u % NBUF]).wait()

        fire_idx(0, 0)

        @pl.loop(0, n_iter)
        def _(t):
            fire_idx(t, 1)
            wait_idx(0)
            for u in range(DEPTH):
                start_g(u)
            for u in range(NU):
                if u == 2 * CK - DEPTH:
                    wait_idx(1)
                wait_g(u)
                start_s(u)
                if u + DEPTH < NU:
                    if u >= 1:
                        wait_s(u - 1)
                    start_g(u + DEPTH)
                if u == 2 * CK + 1:
                    @pl.when(t < n_iter - 1)
                    def _():
                        fire_idx(t + 1, 0)
            for u in range(NU - DEPTH - 1, NU):
                wait_s(u)

        plsc.subcore_barrier()

        @pl.loop(0, per_tile // EL)
        def _(j):
            base = sid * per_tile + j * EL
            pltpu.sync_copy(acc_sh.at[pl.ds(base, EL)], rows_v.at[0])
            pltpu.sync_copy(rows_v.at[0], out_hbm.at[cid].at[pl.ds(base, EL)])

    return k(z_pad, src2d, dst2d)


def _tc_matmul(x, w):
    n, _ = x.shape
    dout = w.shape[0]

    def body(x_ref, w_ref, y_ref):
        y_ref[...] = lax.dot_general(x_ref[...], w_ref[...],
                                     (((1,), (1,)), ((), ())),
                                     preferred_element_type=jnp.float32)

    return pl.pallas_call(
        body,
        out_shape=jax.ShapeDtypeStruct((n, dout), jnp.float32),
    )(x, w)


def _tc_scale(y, deg_parts):
    n, dout = y.shape
    n_pad = deg_parts.shape[1]

    def body(y_ref, dp_ref, z_ref, dis_ref):
        deg = dp_ref[0] + dp_ref[1] + 1.0
        dis = lax.rsqrt(deg)
        disn = dis[:n]
        z_ref[pl.ds(0, n)] = y_ref[...] * disn
        z_ref[pl.ds(n, n_pad - n)] = jnp.zeros((n_pad - n, dout), jnp.float32)
        dis_ref[...] = disn

    return pl.pallas_call(
        body,
        out_shape=(jax.ShapeDtypeStruct((n_pad, dout), jnp.float32),
                   jax.ShapeDtypeStruct((n, 1), jnp.float32)),
    )(y, deg_parts.reshape(NC, n_pad, 1))


def _tc_finish(s_parts, z_pad, dis, bias):
    n = dis.shape[0]
    d = z_pad.shape[1]

    def body(sp_ref, z_ref, dis_ref, b_ref, o_ref):
        s = sp_ref[0][:n] + sp_ref[1][:n] + z_ref[:n]
        o_ref[...] = jnp.maximum(s * dis_ref[...] + b_ref[...], 0.0)

    return pl.pallas_call(
        body,
        out_shape=jax.ShapeDtypeStruct((n, d), jnp.float32),
    )(s_parts, z_pad, dis, bias)


def kernel(x, edge_index, num_nodes, W, bias):
    n, _ = x.shape
    e = edge_index.shape[1]
    per_tile = -(-n // NS)
    per_tile = -(-per_tile // LANES) * LANES
    n_pad = per_tile * NS

    src = edge_index[0].astype(jnp.int32)
    dst = edge_index[1].astype(jnp.int32)

    def dead(k):
        return n + (jnp.arange(k, dtype=jnp.int32) % (n_pad - n))

    unit = LANES * NW * CK * 2
    hist_rows = (-(-2 * e // unit) * unit) // LANES
    hist_pad = hist_rows * LANES - 2 * e
    idx_all = jnp.concatenate([src, dst, dead(hist_pad)]).reshape(hist_rows, LANES)

    EL = 64
    eunit = EL * NW * CK * 2
    e_rows = (-(-e // eunit) * eunit) // EL
    e_pad = e_rows * EL - e
    src2d = jnp.concatenate([src, dead(e_pad)]).reshape(e_rows, EL)
    dst2d = jnp.concatenate([dst, dead(e_pad)]).reshape(e_rows, EL)

    y = _tc_matmul(x, W)
    deg_parts = _sc_degree(idx_all, n_pad)
    z_pad, dis = _tc_scale(y, deg_parts)
    s_parts = _sc_scatter(z_pad, src2d, dst2d, n_pad)
    return _tc_finish(s_parts, z_pad, dis, bias)
